# Initial kernel scaffold; baseline (speedup 1.0000x reference)
import jax
import jax.numpy as jnp
from jax.experimental import pallas as pl


def kernel(predicts, target):
    raise NotImplementedError("write your pallas kernel here")


# speedup vs baseline: 1.5734x; 1.5734x over previous
import jax
import jax.numpy as jnp
from jax.experimental import pallas as pl

_N_VOCAB = 100000
_PAD = 0
_SMOOTHING = 0.1
_CONF = 1.0 - _SMOOTHING
_S = _SMOOTHING / (_N_VOCAB - 2)

_ROWS_BLK = 8


def _loss_kernel(p_ref, t_ref, out_ref):
    i = pl.program_id(0)
    p = p_ref[...]
    t = t_ref[...]
    rowsum = jnp.sum(p, axis=1, keepdims=True)
    col = jax.lax.broadcasted_iota(jnp.int32, p.shape, 1)
    gathered = jnp.sum(jnp.where(col == t, p, 0.0), axis=1, keepdims=True)
    p0 = p[:, 0:1]
    valid = (t != _PAD).astype(jnp.float32)
    c_row = (_N_VOCAB - 2) * _S * jnp.log(_S) + _CONF * jnp.log(_CONF)
    contrib = valid * (c_row - _S * (rowsum - p0) - (_CONF - _S) * gathered)
    partial = jnp.sum(contrib, axis=(0, 1), keepdims=True)

    @pl.when(i == 0)
    def _init():
        out_ref[...] = jnp.zeros_like(out_ref)

    out_ref[...] += partial


def kernel(predicts, target):
    n, v = predicts.shape
    t2 = target.reshape(n, 1).astype(jnp.int32)
    out = pl.pallas_call(
        _loss_kernel,
        grid=(n // _ROWS_BLK,),
        in_specs=[
            pl.BlockSpec((_ROWS_BLK, v), lambda i: (i, 0)),
            pl.BlockSpec((_ROWS_BLK, 1), lambda i: (i, 0)),
        ],
        out_specs=pl.BlockSpec((1, 1), lambda i: (0, 0)),
        out_shape=jax.ShapeDtypeStruct((1, 1), jnp.float32),
    )(predicts, t2)
    return out[0, 0]
